# Initial kernel scaffold; baseline (speedup 1.0000x reference)
import jax
import jax.numpy as jnp
from jax import lax
from jax.experimental import pallas as pl
from jax.experimental.pallas import tpu as pltpu

BB = 16


def _probe_body(z_ref, wi1_ref, o_ref):
    x = z_ref[...]
    w = wi1_ref[...]
    a = jnp.dot(x, w)
    b3 = a.reshape(BB, 6, 64)
    c = jnp.broadcast_to(b3[:, :, None, :], (BB, 6, 6, 64))
    cr = c.reshape(BB, 36, 64)
    bat = lax.dot_general(cr, cr, (((2,), (2,)), ((0,), (0,))))
    red = jnp.sum(c, axis=1)
    cat = jnp.concatenate([red, red], axis=-1)
    s = jnp.sum(bat, axis=(1, 2))[:, None] + jnp.sum(cat, axis=(1, 2))[:, None]
    o_ref[...] = s + x[:, :1]


def kernel(z, Wi1, bi1, Wi2, bi2, Wlink, blink, We0a, We0b, Wc1, bc1, Rc1, cb1,
           Wu1, bu1, We1a, We1b, Wc2, bc2, Rc2, cb2, Wu2, bu2, We2a, We2b,
           Wc4, bc4, Rc4, cb4, Wf0, bf0, Wf1, bf1, We4a, We4b, Wfe0, bfe0,
           Wfe1, bfe1):
    B = z.shape[0]
    out = pl.pallas_call(
        _probe_body,
        grid=(B // BB,),
        in_specs=[
            pl.BlockSpec((BB, 128), lambda i: (i, 0)),
            pl.BlockSpec((128, 384), lambda i: (0, 0)),
        ],
        out_specs=pl.BlockSpec((BB, 1), lambda i: (i, 0)),
        out_shape=jax.ShapeDtypeStruct((B, 1), jnp.float32),
    )(z, Wi1)
    node_out = jnp.zeros((B, 12, 16), jnp.float32) + out[:, :, None]
    edge_out = jnp.zeros((B, 132, 4), jnp.float32) + out[:, :, None]
    return node_out, edge_out


# speedup vs baseline: 4.5699x; 4.5699x over previous
import functools

import jax
import jax.numpy as jnp
import numpy as np
from jax import lax
from jax.experimental import pallas as pl
from jax.experimental.pallas import tpu as pltpu

BB = 16
EH = 64


def _leaky(x):
    return jnp.where(x >= 0, x, 0.05 * x)


def _pair_mask(n, rows):
    r = lax.broadcasted_iota(jnp.int32, (rows, 1), 0)
    p = r % (n * n)
    return jnp.where(p // n != p % n, 1.0, 0.0).astype(jnp.float32)


def _pairs(xf, n, d):
    bb = xf.shape[0]
    xs = jnp.broadcast_to(xf[:, :, None, :], (bb, n, n, d)).reshape(bb * n * n, d)
    xd = jnp.broadcast_to(xf[:, None, :, :], (bb, n, n, d)).reshape(bb * n * n, d)
    return xs, xd


def _edge_attr(xf, n, d, Wa, Wb):
    xs, xd = _pairs(xf, n, d)
    cat = jnp.concatenate([xs, xd], axis=-1)
    return cat, _leaky(_leaky(cat @ Wa) @ Wb)


def _conv(xf, n, din, dout, e_full, Wcr, bcm, R, cb):
    bb = xf.shape[0]
    x2 = xf.reshape(bb * n, din)
    Y = (x2 @ Wcr).reshape(bb * n, EH, dout)
    E3 = e_full.reshape(bb * n, n, EH)
    Z = lax.dot_general(E3, Y, (((2,), (1,)), ((0,), (0,))))
    agg = jnp.sum(Z.reshape(bb, n, n, dout), axis=1).reshape(bb * n, dout)
    xex = (jnp.sum(xf, axis=1, keepdims=True) - xf).reshape(bb * n, din)
    agg = (agg + xex @ bcm) * (1.0 / (n - 1))
    return _leaky(x2 @ R + cb + agg).reshape(bb, n, dout)


def _body(z_ref, wi1_ref, bi1_ref, wi2_ref, bi2_ref, wl_ref, blink_ref,
          we0a_ref, we0b_ref, wc1r_ref, bc1m_ref, rc1_ref, cb1_ref,
          wu1_ref, bu1_ref, we1a_ref, we1b_ref, wc2r_ref, bc2m_ref,
          rc2_ref, cb2_ref, wu2_ref, bu2_ref, we2a_ref, we2b_ref,
          wc4r_ref, bc4m_ref, rc4_ref, cb4_ref, wf0_ref, bf0_ref,
          wf1_ref, bf1_ref, we4a_ref, we4b_ref, wfe0_ref, bfe0_ref,
          wfe1_ref, bfe1_ref, node_ref, edge_ref):
    z = z_ref[...]
    h = _leaky(z @ wi1_ref[...] + bi1_ref[...])
    x0 = _leaky((h @ wi2_ref[...] + bi2_ref[...]).reshape(BB, 3, 64))

    cat0, e0 = _edge_attr(x0, 3, 64, we0a_ref[...], we0b_ref[...])
    gate = jax.nn.sigmoid(
        jnp.sum(cat0 * wl_ref[...], axis=-1, keepdims=True) + blink_ref[...])
    e0 = e0 * (gate * _pair_mask(3, BB * 9))
    x1 = _conv(x0, 3, 64, 64, e0, wc1r_ref[...], bc1m_ref[...],
               rc1_ref[...], cb1_ref[...])

    x1u = (x1.reshape(BB * 3, 64) @ wu1_ref[...] + bu1_ref[...])
    x1u = _leaky(x1u.reshape(BB * 3, 2, 48)).reshape(BB, 6, 48)
    _, e1 = _edge_attr(x1u, 6, 48, we1a_ref[...], we1b_ref[...])
    e1 = e1 * _pair_mask(6, BB * 36)
    x2 = _conv(x1u, 6, 48, 64, e1, wc2r_ref[...], bc2m_ref[...],
               rc2_ref[...], cb2_ref[...])

    x2u = (x2.reshape(BB * 6, 64) @ wu2_ref[...] + bu2_ref[...])
    x2u = _leaky(x2u.reshape(BB * 6, 2, 48)).reshape(BB, 12, 48)
    _, e2 = _edge_attr(x2u, 12, 48, we2a_ref[...], we2b_ref[...])
    e2 = e2 * _pair_mask(12, BB * 144)
    x3 = _conv(x2u, 12, 48, 64, e2, wc4r_ref[...], bc4m_ref[...],
               rc4_ref[...], cb4_ref[...])

    h0 = _leaky(x3.reshape(BB * 12, 64) @ wf0_ref[...] + bf0_ref[...])
    node = h0 @ wf1_ref[...] + bf1_ref[...]
    node_ref[...] = node.reshape(BB, 12, 16)

    _, e4 = _edge_attr(node.reshape(BB, 12, 16), 12, 16,
                       we4a_ref[...], we4b_ref[...])
    hs, hd = _pairs(h0.reshape(BB, 12, 64), 12, 64)
    pairf = 0.5 * (hs + hd)
    ef = _leaky(jnp.concatenate([e2, e4, pairf], axis=-1) @ wfe0_ref[...]
                + bfe0_ref[...])
    edge = ef @ wfe1_ref[...] + bfe1_ref[...]
    edge_ref[...] = edge.reshape(BB, 144, 4)


def _full(shape):
    nd = len(shape)
    return pl.BlockSpec(shape, lambda i: (0,) * nd)


def kernel(z, Wi1, bi1, Wi2, bi2, Wlink, blink, We0a, We0b, Wc1, bc1, Rc1, cb1,
           Wu1, bu1, We1a, We1b, Wc2, bc2, Rc2, cb2, Wu2, bu2, We2a, We2b,
           Wc4, bc4, Rc4, cb4, Wf0, bf0, Wf1, bf1, We4a, We4b, Wfe0, bfe0,
           Wfe1, bfe1):
    B = z.shape[0]

    def relayout(Wc, din, dout):
        return Wc.reshape(EH, din, dout).transpose(1, 0, 2).reshape(din, EH * dout)

    wc1r = relayout(Wc1, 64, 64)
    wc2r = relayout(Wc2, 48, 64)
    wc4r = relayout(Wc4, 48, 64)
    row = lambda v: v.reshape(1, -1)
    ins = [z, Wi1, row(bi1), Wi2, row(bi2), row(Wlink), row(blink),
           We0a, We0b, wc1r, bc1.reshape(64, 64), Rc1, row(cb1),
           Wu1, row(bu1), We1a, We1b, wc2r, bc2.reshape(48, 64),
           Rc2, row(cb2), Wu2, row(bu2), We2a, We2b,
           wc4r, bc4.reshape(48, 64), Rc4, row(cb4), Wf0, row(bf0),
           Wf1, row(bf1), We4a, We4b, Wfe0, row(bfe0), Wfe1, row(bfe1)]

    in_specs = [pl.BlockSpec((BB, 128), lambda i: (i, 0))]
    in_specs += [_full(a.shape) for a in ins[1:]]

    node_out, edge_full = pl.pallas_call(
        _body,
        grid=(B // BB,),
        in_specs=in_specs,
        out_specs=[
            pl.BlockSpec((BB, 12, 16), lambda i: (i, 0, 0)),
            pl.BlockSpec((BB, 144, 4), lambda i: (i, 0, 0)),
        ],
        out_shape=[
            jax.ShapeDtypeStruct((B, 12, 16), jnp.float32),
            jax.ShapeDtypeStruct((B, 144, 4), jnp.float32),
        ],
    )(*ins)

    offdiag = np.array([i * 12 + j for i in range(12) for j in range(12)
                        if i != j], dtype=np.int32)
    return node_out, edge_full[:, offdiag, :]
